# baseline (device time: 21983 ns/iter reference)
import jax
import jax.numpy as jnp
from jax import lax
from jax.experimental import pallas as pl
from jax.experimental.pallas import tpu as pltpu

N_DEV = 4
CAPACITY = 25


def kernel(x, router_W, route_idx, expert_W):
    n_tok, d_model = x.shape
    e_loc, _, d_out = expert_W.shape
    m_out = n_tok // N_DEV
    n_hops = N_DEV - 1

    def body(x_ref, idx_ref, w_ref, out_ref, part_ref, comm_ref,
             send_sems, recv_sems):
        my = lax.axis_index("i")
        left = lax.rem(my + N_DEV - 1, N_DEV)
        right = lax.rem(my + 1, N_DEV)

        barrier_sem = pltpu.get_barrier_semaphore()
        for nbr in (left, right):
            pl.semaphore_signal(barrier_sem, inc=1, device_id=(nbr,),
                                device_id_type=pl.DeviceIdType.MESH)
        pl.semaphore_wait(barrier_sem, 2)

        e_col = idx_ref[:, :]
        r = lax.broadcasted_iota(jnp.int32, (n_tok, n_tok), 0)
        c = lax.broadcasted_iota(jnp.int32, (n_tok, n_tok), 1)
        same = (e_col == idx_ref[:, 0][None, :]) & (c <= r)
        cnt = jnp.sum(same.astype(jnp.float32), axis=1, keepdims=True)
        keep = (cnt <= float(CAPACITY)).astype(jnp.float32)

        acc = jnp.zeros((n_tok, d_out), dtype=jnp.float32)
        for l in range(e_loc):
            glob_e = my * e_loc + l
            mask = keep * (e_col == glob_e).astype(jnp.float32)
            acc = acc + jnp.dot(x_ref[:, :] * mask, w_ref[l],
                                preferred_element_type=jnp.float32)
        part_ref[:, :] = acc

        c0 = lax.rem(my + N_DEV - 1, N_DEV)
        comm_ref[0, :, :] = part_ref[pl.ds(c0 * m_out, m_out), :]
        for s in range(n_hops):
            rdma = pltpu.make_async_remote_copy(
                src_ref=comm_ref.at[s],
                dst_ref=comm_ref.at[s + 1],
                send_sem=send_sems.at[s],
                recv_sem=recv_sems.at[s],
                device_id=(right,),
                device_id_type=pl.DeviceIdType.MESH,
            )
            rdma.start()
            rdma.wait()
            chunk = lax.rem(my + 2 * N_DEV - 2 - s, N_DEV)
            summed = comm_ref[s + 1] + part_ref[pl.ds(chunk * m_out, m_out), :]
            if s < n_hops - 1:
                comm_ref[s + 1, :, :] = summed
            else:
                out_ref[:, :] = summed

    return pl.pallas_call(
        body,
        out_shape=jax.ShapeDtypeStruct((m_out, d_out), jnp.float32),
        in_specs=[
            pl.BlockSpec(memory_space=pltpu.VMEM),
            pl.BlockSpec(memory_space=pltpu.VMEM),
            pl.BlockSpec(memory_space=pltpu.VMEM),
        ],
        out_specs=pl.BlockSpec(memory_space=pltpu.VMEM),
        scratch_shapes=[
            pltpu.VMEM((n_tok, d_out), jnp.float32),
            pltpu.VMEM((N_DEV, m_out, d_out), jnp.float32),
            pltpu.SemaphoreType.DMA((n_hops,)),
            pltpu.SemaphoreType.DMA((n_hops,)),
        ],
        compiler_params=pltpu.CompilerParams(collective_id=0),
    )(x, route_idx, expert_W)


# device time: 15717 ns/iter; 1.3987x vs baseline; 1.3987x over previous
import jax
import jax.numpy as jnp
from jax import lax
from jax.experimental import pallas as pl
from jax.experimental.pallas import tpu as pltpu

N_DEV = 4
CAPACITY = 25


def kernel(x, router_W, route_idx, expert_W):
    n_tok, d_model = x.shape
    e_loc, _, d_out = expert_W.shape
    m_out = n_tok // N_DEV

    def body(x_ref, idx_ref, w_ref, out_ref, keep_ref, send_ref, recv_ref,
             send_sems, recv_sems):
        my = lax.axis_index("i")

        barrier_sem = pltpu.get_barrier_semaphore()
        for d in range(1, N_DEV):
            pl.semaphore_signal(barrier_sem, inc=1,
                                device_id=(lax.rem(my + d, N_DEV),),
                                device_id_type=pl.DeviceIdType.MESH)
        pl.semaphore_wait(barrier_sem, N_DEV - 1)

        e_col = idx_ref[:, :]
        r = lax.broadcasted_iota(jnp.int32, (n_tok, n_tok), 0)
        c = lax.broadcasted_iota(jnp.int32, (n_tok, n_tok), 1)
        same = (e_col == idx_ref[:, 0][None, :]) & (c <= r)
        cnt = jnp.sum(same.astype(jnp.float32), axis=1, keepdims=True)
        keep_ref[:, :] = (cnt <= float(CAPACITY)).astype(jnp.float32)

        def chunk_partial(chunk):
            row0 = chunk * m_out
            xc = x_ref[pl.ds(row0, m_out), :]
            ec = idx_ref[pl.ds(row0, m_out), :]
            kc = keep_ref[pl.ds(row0, m_out), :]
            acc = jnp.zeros((m_out, d_out), dtype=jnp.float32)
            for l in range(e_loc):
                mask = kc * (ec == my * e_loc + l).astype(jnp.float32)
                acc = acc + jnp.dot(xc * mask, w_ref[l],
                                    preferred_element_type=jnp.float32)
            return acc

        rdmas = []
        for d in range(1, N_DEV):
            dest = lax.rem(my + d, N_DEV)
            send_ref[d - 1, :, :] = chunk_partial(dest)
            rdma = pltpu.make_async_remote_copy(
                src_ref=send_ref.at[d - 1],
                dst_ref=recv_ref.at[d - 1],
                send_sem=send_sems.at[d - 1],
                recv_sem=recv_sems.at[d - 1],
                device_id=(dest,),
                device_id_type=pl.DeviceIdType.MESH,
            )
            rdma.start()
            rdmas.append(rdma)

        total = chunk_partial(my)
        for d in range(1, N_DEV):
            rdmas[d - 1].wait_recv()
            total = total + recv_ref[d - 1]
        out_ref[:, :] = total
        for rdma in rdmas:
            rdma.wait_send()

    return pl.pallas_call(
        body,
        out_shape=jax.ShapeDtypeStruct((m_out, d_out), jnp.float32),
        in_specs=[
            pl.BlockSpec(memory_space=pltpu.VMEM),
            pl.BlockSpec(memory_space=pltpu.VMEM),
            pl.BlockSpec(memory_space=pltpu.VMEM),
        ],
        out_specs=pl.BlockSpec(memory_space=pltpu.VMEM),
        scratch_shapes=[
            pltpu.VMEM((n_tok, 1), jnp.float32),
            pltpu.VMEM((N_DEV - 1, m_out, d_out), jnp.float32),
            pltpu.VMEM((N_DEV - 1, m_out, d_out), jnp.float32),
            pltpu.SemaphoreType.DMA((N_DEV - 1,)),
            pltpu.SemaphoreType.DMA((N_DEV - 1,)),
        ],
        compiler_params=pltpu.CompilerParams(collective_id=0),
    )(x, route_idx, expert_W)


# device time: 12905 ns/iter; 1.7034x vs baseline; 1.2179x over previous
import jax
import jax.numpy as jnp
from jax import lax
from jax.experimental import pallas as pl
from jax.experimental.pallas import tpu as pltpu

N_DEV = 4
CAPACITY = 25


def kernel(x, router_W, route_idx, expert_W):
    n_tok, d_model = x.shape
    e_loc, _, d_out = expert_W.shape
    m_out = n_tok // N_DEV

    def body(x_ref, idx_ref, w_ref, out_ref, keep_ref, wbf_ref, send_ref,
             recv_ref, send_sems, recv_sems):
        my = lax.axis_index("i")

        barrier_sem = pltpu.get_barrier_semaphore()
        for d in range(1, N_DEV):
            pl.semaphore_signal(barrier_sem, inc=1,
                                device_id=(lax.rem(my + d, N_DEV),),
                                device_id_type=pl.DeviceIdType.MESH)
        pl.semaphore_wait(barrier_sem, N_DEV - 1)

        e_col = idx_ref[:, :]
        r = lax.broadcasted_iota(jnp.int32, (n_tok, n_tok), 0)
        c = lax.broadcasted_iota(jnp.int32, (n_tok, n_tok), 1)
        same = (e_col == idx_ref[:, 0][None, :]) & (c <= r)
        cnt = jnp.sum(same.astype(jnp.float32), axis=1, keepdims=True)
        keep_ref[:, :] = (cnt <= float(CAPACITY)).astype(jnp.float32)

        wbf_ref[:, :, :] = w_ref[:, :, :].astype(jnp.bfloat16)

        def chunk_partial(chunk):
            row0 = chunk * m_out
            xc = x_ref[pl.ds(row0, m_out), :]
            ec = idx_ref[pl.ds(row0, m_out), :]
            kc = keep_ref[pl.ds(row0, m_out), :]
            acc = jnp.zeros((m_out, d_out), dtype=jnp.float32)
            for l in range(e_loc):
                mask = kc * (ec == my * e_loc + l).astype(jnp.float32)
                acc = acc + jnp.dot((xc * mask).astype(jnp.bfloat16),
                                    wbf_ref[l],
                                    preferred_element_type=jnp.float32)
            return acc

        rdmas = []
        for d in range(1, N_DEV):
            dest = lax.rem(my + d, N_DEV)
            send_ref[d - 1, :, :] = chunk_partial(dest).astype(jnp.bfloat16)
            rdma = pltpu.make_async_remote_copy(
                src_ref=send_ref.at[d - 1],
                dst_ref=recv_ref.at[d - 1],
                send_sem=send_sems.at[d - 1],
                recv_sem=recv_sems.at[d - 1],
                device_id=(dest,),
                device_id_type=pl.DeviceIdType.MESH,
            )
            rdma.start()
            rdmas.append(rdma)

        total = chunk_partial(my)
        for d in range(1, N_DEV):
            rdmas[d - 1].wait_recv()
            total = total + recv_ref[d - 1].astype(jnp.float32)
        out_ref[:, :] = total
        for rdma in rdmas:
            rdma.wait_send()

    return pl.pallas_call(
        body,
        out_shape=jax.ShapeDtypeStruct((m_out, d_out), jnp.float32),
        in_specs=[
            pl.BlockSpec(memory_space=pltpu.VMEM),
            pl.BlockSpec(memory_space=pltpu.VMEM),
            pl.BlockSpec(memory_space=pltpu.VMEM),
        ],
        out_specs=pl.BlockSpec(memory_space=pltpu.VMEM),
        scratch_shapes=[
            pltpu.VMEM((n_tok, 1), jnp.float32),
            pltpu.VMEM((e_loc, d_model, d_out), jnp.bfloat16),
            pltpu.VMEM((N_DEV - 1, m_out, d_out), jnp.bfloat16),
            pltpu.VMEM((N_DEV - 1, m_out, d_out), jnp.bfloat16),
            pltpu.SemaphoreType.DMA((N_DEV - 1,)),
            pltpu.SemaphoreType.DMA((N_DEV - 1,)),
        ],
        compiler_params=pltpu.CompilerParams(collective_id=0),
    )(x, route_idx, expert_W)
